# Initial kernel scaffold; baseline (speedup 1.0000x reference)
import jax
import jax.numpy as jnp
from jax.experimental import pallas as pl


def kernel(pred_sid, pred_corr_factor, rechit_energy, no_noise_idx, pred_beta, is_track):
    raise NotImplementedError("write your pallas kernel here")


# speedup vs baseline: 10.3578x; 10.3578x over previous
import functools

import jax
import jax.numpy as jnp
from jax import lax
from jax.experimental import pallas as pl
from jax.experimental.pallas import tpu as pltpu
from jax.experimental.pallas import tpu_sc as plsc

N_FILT = 150000
N_ORIG = 200000
NC, NS, L = 2, 16, 16
NW = NC * NS
RB = 128
ROWS = 37
C = ROWS * RB
NP = NW * C
NBINS = 256
VPW = C // L

_mesh = plsc.VectorSubcoreMesh(core_axis_name="c", subcore_axis_name="s")


def _wid():
    return lax.axis_index("s") * NC + lax.axis_index("c")


@functools.partial(
    pl.kernel,
    out_type=jax.ShapeDtypeStruct((NW, NBINS), jnp.float32),
    mesh=_mesh,
    scratch_types=[
        pltpu.VMEM((ROWS, RB), jnp.int32),
        pltpu.VMEM((C,), jnp.int32),
        pltpu.VMEM((C,), jnp.float32),
        pltpu.VMEM((C,), jnp.float32),
        pltpu.VMEM((C,), jnp.int32),
        pltpu.VMEM((NBINS,), jnp.float32),
        pltpu.SemaphoreType.DMA,
    ],
    compiler_params=pltpu.CompilerParams(needs_layout_passes=False),
)
def _accumulate(idx_hbm, sid_hbm, corr_hbm, energy_hbm, istrack_hbm,
                partials_hbm, idx_v, sid_v, corr_v, e_v, t_v, acc_v, sem):
    w = _wid()
    pltpu.sync_copy(idx_hbm.at[w], idx_v)

    def fire(j, carry):
        pltpu.make_async_copy(
            energy_hbm.at[idx_v.at[j]], e_v.at[pl.ds(j * jnp.int32(RB), RB)], sem
        ).start()
        pltpu.make_async_copy(
            istrack_hbm.at[idx_v.at[j]], t_v.at[pl.ds(j * jnp.int32(RB), RB)], sem
        ).start()
        return carry

    lax.fori_loop(jnp.int32(0), jnp.int32(ROWS), fire, 0)

    pltpu.sync_copy(sid_hbm.at[w], sid_v)
    pltpu.sync_copy(corr_hbm.at[w], corr_v)

    zeros = jnp.zeros((L,), jnp.float32)

    def zero_body(k, carry):
        acc_v[pl.ds(k * jnp.int32(L), L)] = zeros
        return carry

    lax.fori_loop(jnp.int32(0), jnp.int32(NBINS // L), zero_body, 0)

    pltpu.make_async_copy(energy_hbm.at[pl.ds(0, C)], e_v, sem).wait()
    pltpu.make_async_copy(istrack_hbm.at[pl.ds(0, C)], t_v, sem).wait()

    def body(i, carry):
        o = i * jnp.int32(L)
        s = sid_v[pl.ds(o, L)]
        e = e_v[pl.ds(o, L)]
        t = t_v[pl.ds(o, L)]
        cf = corr_v[pl.ds(o, L)]
        val = jnp.where(t == 0, e * cf, zeros)
        plsc.addupdate_scatter(acc_v, [s], val)
        return carry

    lax.fori_loop(jnp.int32(0), jnp.int32(VPW), body, 0)
    pltpu.sync_copy(acc_v, partials_hbm.at[w])


@functools.partial(
    pl.kernel,
    out_type=jax.ShapeDtypeStruct((NW, C), jnp.float32),
    mesh=_mesh,
    scratch_types=[
        pltpu.VMEM((NW * NBINS,), jnp.float32),
        pltpu.VMEM((NBINS,), jnp.float32),
        pltpu.VMEM((C,), jnp.int32),
        pltpu.VMEM((C,), jnp.float32),
    ],
    compiler_params=pltpu.CompilerParams(needs_layout_passes=False),
)
def _gather_back(partials_hbm, sid_hbm, out_hbm, pall_v, s_v, sid_v, out_v):
    w = _wid()
    pltpu.sync_copy(partials_hbm, pall_v)
    pltpu.sync_copy(sid_hbm.at[w], sid_v)

    def red(k, carry):
        o = k * jnp.int32(L)
        acc = pall_v[pl.ds(o, L)]
        for r in range(1, NW):
            acc = acc + pall_v[pl.ds(jnp.int32(r * NBINS) + o, L)]
        s_v[pl.ds(o, L)] = acc
        return carry

    lax.fori_loop(jnp.int32(0), jnp.int32(NBINS // L), red, 0)

    def gat(i, carry):
        o = i * jnp.int32(L)
        s = sid_v[pl.ds(o, L)]
        out_v[pl.ds(o, L)] = plsc.load_gather(s_v, [s])
        return carry

    lax.fori_loop(jnp.int32(0), jnp.int32(VPW), gat, 0)
    pltpu.sync_copy(out_v, out_hbm.at[w])


def kernel(pred_sid, pred_corr_factor, rechit_energy, no_noise_idx,
           pred_beta, is_track):
    del pred_beta
    idx = no_noise_idx[:, 0].astype(jnp.int32)
    sid = pred_sid[:, 0].astype(jnp.int32)
    corr = pred_corr_factor[:, 0].astype(jnp.float32)
    energy = rechit_energy[:, 0].astype(jnp.float32)
    istrack = is_track[:, 0].astype(jnp.int32)

    pad = NP - N_FILT
    idx_p = jnp.pad(idx, (0, pad)).reshape(NW, ROWS, RB)
    sid_p = jnp.pad(sid, (0, pad)).reshape(NW, C)
    corr_p = jnp.pad(corr, (0, pad)).reshape(NW, C)

    partials = _accumulate(idx_p, sid_p, corr_p, energy, istrack)
    out = _gather_back(partials.reshape(-1), sid_p)
    return out.reshape(-1)[:N_FILT, None]


# speedup vs baseline: 11.0700x; 1.0688x over previous
import functools

import jax
import jax.numpy as jnp
from jax import lax
from jax.experimental import pallas as pl
from jax.experimental.pallas import tpu as pltpu
from jax.experimental.pallas import tpu_sc as plsc

N_FILT = 150000
N_ORIG = 200000
NC, NS, L = 2, 16, 16
RB = 128
ROWS = 74
CA = ROWS * RB
NP = NS * CA
CG = CA // NC
NBINS = 256
I32 = jnp.int32

_mesh = plsc.VectorSubcoreMesh(core_axis_name="c", subcore_axis_name="s")


@functools.partial(
    pl.kernel,
    out_type=jax.ShapeDtypeStruct((NP,), jnp.float32),
    mesh=_mesh,
    scratch_types=[
        pltpu.VMEM((ROWS, RB), jnp.int32),
        pltpu.VMEM((CA,), jnp.int32),
        pltpu.VMEM((CA,), jnp.float32),
        pltpu.VMEM((CA,), jnp.int32),
        pltpu.VMEM((NBINS,), jnp.float32),
        pltpu.VMEM((NS * NBINS,), jnp.float32),
        pltpu.VMEM((NBINS,), jnp.float32),
        pltpu.VMEM((CG,), jnp.float32),
        pltpu.VMEM_SHARED((NS * NBINS,), jnp.float32),
        pltpu.SemaphoreType.DMA,
    ],
    compiler_params=pltpu.CompilerParams(needs_layout_passes=False),
)
def _oc_gather_energy(idx_hbm, sid_hbm, tbl_hbm, corr_hbm, out_hbm,
                      idx_v, sid_v, corr_v, epk, acc_v, pall_v, s_v, out_v,
                      shared, sem):
    c = lax.axis_index("c")
    s = lax.axis_index("s")
    pltpu.sync_copy(idx_hbm.at[s], idx_v)

    def fire(j, carry):
        pltpu.make_async_copy(
            tbl_hbm.at[idx_v.at[j]], epk.at[pl.ds(j * I32(RB), RB)], sem
        ).start()
        return carry

    lax.fori_loop(I32(0), I32(ROWS), fire, 0)

    pltpu.sync_copy(sid_hbm.at[pl.ds(s * I32(CA), CA)], sid_v)
    pltpu.sync_copy(corr_hbm.at[pl.ds(s * I32(CA), CA)], corr_v)

    zeros = jnp.zeros((L,), jnp.float32)
    iota = lax.iota(jnp.int32, L)

    def zero_body(k, carry):
        acc_v[pl.ds(k * I32(L), L)] = zeros
        return carry

    lax.fori_loop(I32(0), I32(NBINS // L), zero_body, 0)

    pltpu.make_async_copy(tbl_hbm.at[pl.ds(0, CA)], epk, sem).wait()

    ones = jnp.ones((L,), jnp.int32)

    def body(i, carry):
        o = i * I32(L)
        sg = sid_v[pl.ds(o, L)]
        ev = epk[pl.ds(o, L)]
        t = jnp.bitwise_and(ev, ones)
        e = plsc.bitcast(jnp.bitwise_and(ev, ~ones), jnp.float32)
        cf = corr_v[pl.ds(o, L)]
        val = jnp.where(t == 0, e * cf, zeros)
        plsc.addupdate_scatter(acc_v, [sg], val)
        return carry

    lax.fori_loop(I32(0), I32(CA // L), body, 0)

    pltpu.sync_copy(acc_v, shared.at[pl.ds(s * I32(NBINS), NBINS)])
    plsc.subcore_barrier()
    pltpu.sync_copy(shared, pall_v)

    def red(k, carry):
        o = k * I32(L)
        acc = pall_v[pl.ds(o, L)]
        for r in range(1, NS):
            acc = acc + pall_v[pl.ds(I32(r * NBINS) + o, L)]
        s_v[pl.ds(o, L)] = acc
        return carry

    lax.fori_loop(I32(0), I32(NBINS // L), red, 0)

    gbase = c * I32(CG)

    def gat(i, carry):
        o = i * I32(L)
        sg = sid_v[pl.ds(gbase + o, L)]
        out_v[pl.ds(o, L)] = plsc.load_gather(s_v, [sg])
        return carry

    lax.fori_loop(I32(0), I32(CG // L), gat, 0)
    pltpu.sync_copy(out_v, out_hbm.at[pl.ds(s * I32(CA) + gbase, CG)])


def kernel(pred_sid, pred_corr_factor, rechit_energy, no_noise_idx,
           pred_beta, is_track):
    del pred_beta
    idx = no_noise_idx[:, 0].astype(jnp.int32)
    sid = pred_sid[:, 0].astype(jnp.int32)
    corr = pred_corr_factor[:, 0].astype(jnp.float32)
    energy = rechit_energy[:, 0].astype(jnp.float32)
    istrack = is_track[:, 0].astype(jnp.int32)
    tbl = jnp.bitwise_or(
        jnp.bitwise_and(energy.view(jnp.int32), jnp.int32(-2)), istrack)

    pad = NP - N_FILT
    idx_p = jnp.pad(idx, (0, pad)).reshape(NS, ROWS, RB)
    sid_p = jnp.pad(sid, (0, pad))
    corr_p = jnp.pad(corr, (0, pad))

    out = _oc_gather_energy(idx_p, sid_p, tbl, corr_p)
    return out[:N_FILT, None]
